# Initial kernel scaffold; baseline (speedup 1.0000x reference)
import jax
import jax.numpy as jnp
from jax.experimental import pallas as pl


def kernel(x, Wf1, Wd1, Wf2, Wd2, Wf3, Wd3, Wf4, Wd4, Wf5, Wd5, Ws1f, Ws1d, Ws2f, Ws2d, Wstd, W1, b1, W2, b2, W3, b3, equiv, mix):
    raise NotImplementedError("write your pallas kernel here")


# speedup vs baseline: 1.3397x; 1.3397x over previous
import functools

import jax
import jax.numpy as jnp
from jax.experimental import pallas as pl

EPS = 1e-6
NS = 0.2
K = 20
ROWS = 256


def _topk_kernel(pd_ref, idx_ref, *, n):
    v = pd_ref[0]
    iota = jax.lax.broadcasted_iota(jnp.int32, v.shape, 1)
    neg = jnp.float32(-jnp.inf)
    for j in range(K):
        m = jnp.max(v, axis=1, keepdims=True)
        cand = jnp.where(v == m, iota, n)
        ij = jnp.min(cand, axis=1)
        idx_ref[0, :, j] = ij
        v = jnp.where(iota == ij[:, None], neg, v)


def _knn(xf, k):
    inner = -2.0 * jnp.einsum('bcn,bcm->bnm', xf, xf)
    xx = jnp.sum(xf * xf, axis=1, keepdims=True)
    pd = -xx - inner - jnp.transpose(xx, (0, 2, 1))
    b, n, _ = pd.shape
    return pl.pallas_call(
        functools.partial(_topk_kernel, n=n),
        grid=(b, n // ROWS),
        in_specs=[pl.BlockSpec((1, ROWS, n), lambda i, r: (i, r, 0))],
        out_specs=pl.BlockSpec((1, ROWS, K), lambda i, r: (i, r, 0)),
        out_shape=jax.ShapeDtypeStruct((b, n, K), jnp.int32),
    )(pd)


def _get_graph_feature(x, k):
    b, nf, _, n = x.shape
    xf = x.reshape(b, nf * 3, n)
    idx = _knn(xf, k)
    xt = jnp.transpose(x, (0, 3, 1, 2))
    feat = jax.vmap(lambda xb, ib: xb[ib])(xt, idx)
    xc = xt[:, :, None, :, :]
    out = jnp.concatenate([feat - xc, jnp.broadcast_to(xc, feat.shape)], axis=3)
    return jnp.transpose(out, (0, 3, 4, 1, 2))


def _vn_bn(x):
    norm = jnp.sqrt(jnp.sum(x * x, axis=2)) + EPS
    axes = (0,) + tuple(range(2, norm.ndim))
    mean = jnp.mean(norm, axis=axes, keepdims=True)
    var = jnp.var(norm, axis=axes, keepdims=True)
    nbn = (norm - mean) / jnp.sqrt(var + 1e-5)
    return x / jnp.expand_dims(norm, 2) * jnp.expand_dims(nbn, 2)


def _vn_lrelu(x, Wf, Wd):
    p = jnp.einsum('oi,bi...->bo...', Wf, x)
    p = _vn_bn(p)
    d = jnp.einsum('oi,bi...->bo...', Wd, x)
    dot = jnp.sum(p * d, axis=2, keepdims=True)
    mask = (dot >= 0).astype(x.dtype)
    dns = jnp.sum(d * d, axis=2, keepdims=True)
    return NS * p + (1.0 - NS) * (mask * p + (1.0 - mask) * (p - dot / (dns + EPS) * d))


def _bn_flat(x):
    m = jnp.mean(x, axis=0, keepdims=True)
    v = jnp.var(x, axis=0, keepdims=True)
    return (x - m) / jnp.sqrt(v + 1e-5)


def kernel(x, Wf1, Wd1, Wf2, Wd2, Wf3, Wd3, Wf4, Wd4, Wf5, Wd5, Ws1f, Ws1d, Ws2f, Ws2d, Wstd, W1, b1, W2, b2, W3, b3, equiv, mix):
    b, _, n = x.shape
    h = x[:, None, :, :]
    h = _get_graph_feature(h, K)
    h = _vn_lrelu(h, Wf1, Wd1)
    x1 = jnp.mean(h, axis=-1)
    h = _get_graph_feature(x1, K)
    h = _vn_lrelu(h, Wf2, Wd2)
    x2 = jnp.mean(h, axis=-1)
    h = _get_graph_feature(x2, K)
    h = _vn_lrelu(h, Wf3, Wd3)
    x3 = jnp.mean(h, axis=-1)
    h = _get_graph_feature(x3, K)
    h = _vn_lrelu(h, Wf4, Wd4)
    x4 = jnp.mean(h, axis=-1)
    h = jnp.concatenate([x1, x2, x3, x4], axis=1)
    h = _vn_lrelu(h, Wf5, Wd5)
    hm = jnp.broadcast_to(jnp.mean(h, axis=-1, keepdims=True), h.shape)
    h = jnp.concatenate([h, hm], axis=1)
    z = _vn_lrelu(h, Ws1f, Ws1d)
    z = _vn_lrelu(z, Ws2f, Ws2d)
    z = jnp.einsum('oi,bi...->bo...', Wstd, z)
    z = jnp.swapaxes(z, 1, 2)
    hs = jnp.einsum('bijm,bjkm->bikm', h, z)
    hs = hs.reshape(b, -1, n)
    f1 = jnp.max(hs, axis=-1)
    f2 = jnp.mean(hs, axis=-1)
    f = jnp.concatenate([f1, f2], axis=1)
    f = jax.nn.leaky_relu(_bn_flat(f @ W1.T + b1), NS)
    f = jax.nn.leaky_relu(_bn_flat(f @ W2.T + b2), NS)
    return f @ W3.T + b3


# speedup vs baseline: 1.3574x; 1.0132x over previous
import functools

import jax
import jax.numpy as jnp
from jax.experimental import pallas as pl

EPS = 1e-6
NS = 0.2
K = 20
ROWS = 512


def _topk_kernel(pd_ref, idx_ref, *, n):
    v = pd_ref[0]
    iota = jax.lax.broadcasted_iota(jnp.int32, v.shape, 1)
    neg = jnp.float32(-jnp.inf)
    for j in range(K):
        m = jnp.max(v, axis=1, keepdims=True)
        cand = jnp.where(v == m, iota, n)
        ij = jnp.min(cand, axis=1)
        idx_ref[0, :, j] = ij
        v = jnp.where(iota == ij[:, None], neg, v)


def _knn(xf, k):
    inner = -2.0 * jnp.einsum('bcn,bcm->bnm', xf, xf)
    xx = jnp.sum(xf * xf, axis=1, keepdims=True)
    pd = -xx - inner - jnp.transpose(xx, (0, 2, 1))
    b, n, _ = pd.shape
    return pl.pallas_call(
        functools.partial(_topk_kernel, n=n),
        grid=(b, n // ROWS),
        in_specs=[pl.BlockSpec((1, ROWS, n), lambda i, r: (i, r, 0))],
        out_specs=pl.BlockSpec((1, ROWS, K), lambda i, r: (i, r, 0)),
        out_shape=jax.ShapeDtypeStruct((b, n, K), jnp.int32),
    )(pd)


def _get_graph_feature(x, k):
    b, nf, _, n = x.shape
    xf = x.reshape(b, nf * 3, n)
    idx = _knn(xf, k)
    xt = jnp.transpose(x, (0, 3, 1, 2))
    feat = jax.vmap(lambda xb, ib: xb[ib])(xt, idx)
    xc = xt[:, :, None, :, :]
    out = jnp.concatenate([feat - xc, jnp.broadcast_to(xc, feat.shape)], axis=3)
    return jnp.transpose(out, (0, 3, 4, 1, 2))


def _vn_bn(x):
    norm = jnp.sqrt(jnp.sum(x * x, axis=2)) + EPS
    axes = (0,) + tuple(range(2, norm.ndim))
    mean = jnp.mean(norm, axis=axes, keepdims=True)
    var = jnp.var(norm, axis=axes, keepdims=True)
    nbn = (norm - mean) / jnp.sqrt(var + 1e-5)
    return x / jnp.expand_dims(norm, 2) * jnp.expand_dims(nbn, 2)


def _vn_lrelu(x, Wf, Wd):
    p = jnp.einsum('oi,bi...->bo...', Wf, x)
    p = _vn_bn(p)
    d = jnp.einsum('oi,bi...->bo...', Wd, x)
    dot = jnp.sum(p * d, axis=2, keepdims=True)
    mask = (dot >= 0).astype(x.dtype)
    dns = jnp.sum(d * d, axis=2, keepdims=True)
    return NS * p + (1.0 - NS) * (mask * p + (1.0 - mask) * (p - dot / (dns + EPS) * d))


def _bn_flat(x):
    m = jnp.mean(x, axis=0, keepdims=True)
    v = jnp.var(x, axis=0, keepdims=True)
    return (x - m) / jnp.sqrt(v + 1e-5)


def kernel(x, Wf1, Wd1, Wf2, Wd2, Wf3, Wd3, Wf4, Wd4, Wf5, Wd5, Ws1f, Ws1d, Ws2f, Ws2d, Wstd, W1, b1, W2, b2, W3, b3, equiv, mix):
    b, _, n = x.shape
    h = x[:, None, :, :]
    h = _get_graph_feature(h, K)
    h = _vn_lrelu(h, Wf1, Wd1)
    x1 = jnp.mean(h, axis=-1)
    h = _get_graph_feature(x1, K)
    h = _vn_lrelu(h, Wf2, Wd2)
    x2 = jnp.mean(h, axis=-1)
    h = _get_graph_feature(x2, K)
    h = _vn_lrelu(h, Wf3, Wd3)
    x3 = jnp.mean(h, axis=-1)
    h = _get_graph_feature(x3, K)
    h = _vn_lrelu(h, Wf4, Wd4)
    x4 = jnp.mean(h, axis=-1)
    h = jnp.concatenate([x1, x2, x3, x4], axis=1)
    h = _vn_lrelu(h, Wf5, Wd5)
    hm = jnp.broadcast_to(jnp.mean(h, axis=-1, keepdims=True), h.shape)
    h = jnp.concatenate([h, hm], axis=1)
    z = _vn_lrelu(h, Ws1f, Ws1d)
    z = _vn_lrelu(z, Ws2f, Ws2d)
    z = jnp.einsum('oi,bi...->bo...', Wstd, z)
    z = jnp.swapaxes(z, 1, 2)
    hs = jnp.einsum('bijm,bjkm->bikm', h, z)
    hs = hs.reshape(b, -1, n)
    f1 = jnp.max(hs, axis=-1)
    f2 = jnp.mean(hs, axis=-1)
    f = jnp.concatenate([f1, f2], axis=1)
    f = jax.nn.leaky_relu(_bn_flat(f @ W1.T + b1), NS)
    f = jax.nn.leaky_relu(_bn_flat(f @ W2.T + b2), NS)
    return f @ W3.T + b3
